# baseline (speedup 1.0000x reference)
import jax
import jax.numpy as jnp
from jax import lax
from jax.experimental import pallas as pl
from jax.experimental.pallas import tpu as pltpu
from jax.experimental.pallas import tpu_sc as plsc

BATCH = 4096
HIST = 50
DIM = 64
NUM_CORES = 2
NUM_SUBCORES = 16
NW = NUM_CORES * NUM_SUBCORES
ROWS_W = BATCH // NW
G = 8
NGRP = ROWS_W // G


def _emb_body(idx_hbm, table_hbm, out_hbm, idx_v, rows_a, rows_b,
              gs_a, gs_b, os_a, os_b):
    wid = lax.axis_index("s") * NUM_CORES + lax.axis_index("c")
    base = wid * ROWS_W

    pltpu.sync_copy(idx_hbm.at[pl.ds(base, ROWS_W)], idx_v)

    bufs = (rows_a, rows_b)
    gsems = (gs_a, gs_b)
    osems = (os_a, os_b)

    def fire_group(g, buf, sem):
        handles = []
        for c in range(G):
            r = g * G + c
            handles.append(pltpu.async_copy(
                table_hbm.at[idx_v.at[r]], buf.at[c], sem))
        return handles

    gh = [fire_group(0, bufs[0], gsems[0]), None]
    oh = [None, None]
    for g in range(NGRP):
        cur = g % 2
        nxt = 1 - cur
        if g + 1 < NGRP:
            if oh[nxt] is not None:
                oh[nxt].wait()
            gh[nxt] = fire_group(g + 1, bufs[nxt], gsems[nxt])
        for h in gh[cur]:
            h.wait()
        oh[cur] = pltpu.async_copy(
            bufs[cur], out_hbm.at[pl.ds(base + g * G, G)], osems[cur])
    oh[0].wait()
    oh[1].wait()


def kernel(input, weight):
    mesh = plsc.VectorSubcoreMesh(core_axis_name="c", subcore_axis_name="s")
    return pl.kernel(
        _emb_body,
        out_type=jax.ShapeDtypeStruct((BATCH, HIST, DIM), jnp.float32),
        mesh=mesh,
        scratch_types=[
            pltpu.VMEM((ROWS_W, HIST), jnp.int32),
            pltpu.VMEM((G, HIST, DIM), jnp.float32),
            pltpu.VMEM((G, HIST, DIM), jnp.float32),
            pltpu.SemaphoreType.DMA,
            pltpu.SemaphoreType.DMA,
            pltpu.SemaphoreType.DMA,
            pltpu.SemaphoreType.DMA,
        ],
        compiler_params=pltpu.CompilerParams(use_tc_tiling_on_sc=False),
    )(input, weight)


# speedup vs baseline: 1.0455x; 1.0455x over previous
import jax
import jax.numpy as jnp
from jax import lax
from jax.experimental import pallas as pl
from jax.experimental.pallas import tpu as pltpu
from jax.experimental.pallas import tpu_sc as plsc

BATCH = 4096
HIST = 50
DIM = 64
VOCAB = 100000
NUM_CORES = 2
NUM_SUBCORES = 16
NW = NUM_CORES * NUM_SUBCORES
FEAT_W = DIM // NW
NCHUNK = BATCH // 16
UNROLL = 8


def _emb_body(idx_hbm, table_hbm, out_hbm, ish, wrow,
              ib0, ib1, ob0, ob1, is0, is1, os0, os1):
    cid = lax.axis_index("c")
    sid = lax.axis_index("s")
    w = sid * NUM_CORES + cid

    @pl.when(sid == 0)
    def _():
        pltpu.sync_copy(idx_hbm, ish)
    plsc.subcore_barrier()

    ibufs = (ib0, ib1)
    obufs = (ob0, ob1)
    isems = (is0, is1)
    osems = (os0, os1)
    ih = [None, None]
    oh = [None, None]

    for dd in range(FEAT_W):
        d = FEAT_W * w + dd
        pltpu.sync_copy(table_hbm.at[d], wrow)
        ih[0] = pltpu.async_copy(ish.at[0], ibufs[0], isems[0])
        for h in range(HIST):
            par = h % 2
            if h + 1 < HIST:
                ih[1 - par] = pltpu.async_copy(
                    ish.at[h + 1], ibufs[1 - par], isems[1 - par])
            ih[par].wait()
            if oh[par] is not None:
                oh[par].wait()
            idx_v = ibufs[par]
            out_v = obufs[par]

            def chunk(cc, _, idx_v=idx_v, out_v=out_v):
                for u in range(UNROLL):
                    off = (cc * UNROLL + u) * 16
                    i16 = idx_v[pl.ds(off, 16)]
                    out_v[pl.ds(off, 16)] = plsc.load_gather(wrow, [i16])
                return ()

            lax.fori_loop(0, NCHUNK // UNROLL, chunk, ())
            oh[par] = pltpu.async_copy(out_v, out_hbm.at[h, d], osems[par])
    oh[0].wait()
    oh[1].wait()


def kernel(input, weight):
    mesh = plsc.VectorSubcoreMesh(core_axis_name="c", subcore_axis_name="s")
    out_t = pl.kernel(
        _emb_body,
        out_type=jax.ShapeDtypeStruct((HIST, DIM, BATCH), jnp.float32),
        mesh=mesh,
        scratch_types=[
            pltpu.VMEM_SHARED((HIST, BATCH), jnp.int32),
            pltpu.VMEM((VOCAB,), jnp.float32),
            pltpu.VMEM((BATCH,), jnp.int32),
            pltpu.VMEM((BATCH,), jnp.int32),
            pltpu.VMEM((BATCH,), jnp.float32),
            pltpu.VMEM((BATCH,), jnp.float32),
            pltpu.SemaphoreType.DMA,
            pltpu.SemaphoreType.DMA,
            pltpu.SemaphoreType.DMA,
            pltpu.SemaphoreType.DMA,
        ],
        compiler_params=pltpu.CompilerParams(use_tc_tiling_on_sc=False,
                                             needs_layout_passes=False),
    )(input.T, weight.T)
    return out_t.transpose(2, 0, 1)
